# baseline (speedup 1.0000x reference)
import functools

import jax
import jax.numpy as jnp
from jax import lax
from jax.experimental import pallas as pl
from jax.experimental.pallas import tpu as pltpu
from jax.experimental.pallas import tpu_sc as plsc

_DIM = 16
_BATCH = 16384
_NC = 2
_NS = 16
_NW = _NC * _NS
_EDGES_PER_W = _BATCH // _NW
_ROWS_PER_W = 2 * _EDGES_PER_W
_CHUNK = 128
_NCHUNK = _ROWS_PER_W // _CHUNK
_GROUPS = _EDGES_PER_W // _DIM


def _rsqrt_scale(x):
    i = lax.bitcast_convert_type(x, jnp.int32)
    i = jnp.int32(0x5F3759DF) - lax.shift_right_arithmetic(
        i, jnp.full((_DIM,), 1, jnp.int32))
    y = lax.bitcast_convert_type(i, jnp.float32)
    for _ in range(3):
        y = y * (1.5 - 0.5 * x * y * y)
    return jnp.where(x > 1.0, y, jnp.full((_DIM,), 1.0, jnp.float32))


def _decoder_body(idx_hbm, table_hbm, out_hbm, idx_v, rows_v, probs_v, sem):
    wid = lax.axis_index("s") * _NC + lax.axis_index("c")

    pltpu.sync_copy(idx_hbm.at[pl.ds(wid * _NCHUNK, _NCHUNK), :], idx_v)

    copies = []
    for k in range(_NCHUNK):
        copies.append(
            pltpu.async_copy(
                table_hbm.at[idx_v.at[k]],
                rows_v.at[pl.ds(k * _CHUNK, _CHUNK), :],
                sem,
            ))
    for cp in copies:
        cp.wait()

    lane = lax.iota(jnp.int32, _DIM)

    def group(g, carry):
        base = g * (2 * _DIM)
        ss = jnp.zeros((_DIM,), jnp.float32)
        dd = jnp.zeros((_DIM,), jnp.float32)
        sd = jnp.zeros((_DIM,), jnp.float32)
        for e in range(_DIM):
            s = rows_v[base + 2 * e, :]
            d = rows_v[base + 2 * e + 1, :]
            m = lane == e
            ss = jnp.where(m, jnp.sum(s * s), ss)
            dd = jnp.where(m, jnp.sum(d * d), dd)
            sd = jnp.where(m, jnp.sum(s * d), sd)
        prod = sd * _rsqrt_scale(ss) * _rsqrt_scale(dd)
        probs_v[pl.ds(g * _DIM, _DIM)] = 1.0 / (1.0 + jnp.exp(-prod))
        return carry

    lax.fori_loop(0, _GROUPS, group, 0)

    pltpu.sync_copy(probs_v, out_hbm.at[pl.ds(wid * _EDGES_PER_W, _EDGES_PER_W)])


@jax.jit
def _decoder(idx2, table):
    mesh = plsc.VectorSubcoreMesh(core_axis_name="c", subcore_axis_name="s")
    return pl.kernel(
        _decoder_body,
        mesh=mesh,
        compiler_params=pltpu.CompilerParams(
            needs_layout_passes=False, use_tc_tiling_on_sc=False),
        out_type=jax.ShapeDtypeStruct((_BATCH,), jnp.float32),
        scratch_types=[
            pltpu.VMEM((_NCHUNK, _CHUNK), jnp.int32),
            pltpu.VMEM((_ROWS_PER_W, _DIM), jnp.float32),
            pltpu.VMEM((_EDGES_PER_W,), jnp.float32),
            pltpu.SemaphoreType.DMA,
        ],
    )(idx2, table)


def kernel(edges, table):
    idx2 = edges.astype(jnp.int32).reshape(_BATCH * 2 // _CHUNK, _CHUNK)
    return _decoder(idx2, table)
